# baseline (speedup 1.0000x reference)
import functools

import jax
import jax.numpy as jnp
from jax import lax
from jax.experimental import pallas as pl
from jax.experimental.pallas import tpu as pltpu
from jax.experimental.pallas import tpu_sc as plsc

_VOCAB = 257216
_EMBED = 2304
_NC = 2
_NS = 16
_NW = _NC * _NS


def _build_gather(batch: int):
    b_per_w = batch // _NW
    chunk = 8
    n_chunk = b_per_w // chunk
    nbuf = 4

    mesh = plsc.VectorSubcoreMesh(core_axis_name="c", subcore_axis_name="s")

    @functools.partial(
        pl.kernel,
        mesh=mesh,
        out_type=jax.ShapeDtypeStruct((batch, _EMBED), jnp.float32),
        scratch_types=[
            pltpu.VMEM((b_per_w,), jnp.int32),
            pltpu.VMEM_SHARED((_NS, 2, chunk, _EMBED), jnp.float32),
        ] + [pltpu.VMEM((chunk, _EMBED), jnp.float32)] * nbuf
          + [pltpu.SemaphoreType.DMA] * (2 * nbuf + 2),
    )
    def gather_kernel(idx_hbm, table_hbm, out_hbm, idx_v, spmem,
                      *bufs_and_sems):
        bufs = bufs_and_sems[:nbuf]
        gsems = bufs_and_sems[nbuf:2 * nbuf]
        csems = bufs_and_sems[2 * nbuf:3 * nbuf]
        ssems = bufs_and_sems[3 * nbuf:]
        sid = lax.axis_index("s")
        wid = sid * _NC + lax.axis_index("c")
        base = wid * b_per_w
        pltpu.sync_copy(idx_hbm.at[pl.ds(base, b_per_w)], idx_v)

        def g_start(i, b):
            off = pl.multiple_of(i * chunk, 8)
            pltpu.async_copy(
                table_hbm.at[idx_v.at[pl.ds(off, chunk)]], bufs[b], gsems[b])

        def g_wait(i, b):
            off = pl.multiple_of(i * chunk, 8)
            pltpu.make_async_copy(
                table_hbm.at[idx_v.at[pl.ds(off, chunk)]], bufs[b],
                gsems[b]).wait()

        def c_start(b, p):
            pltpu.async_copy(bufs[b], spmem.at[sid, p], csems[b])

        def c_wait(b, p):
            pltpu.make_async_copy(bufs[b], spmem.at[sid, p], csems[b]).wait()

        def s_start(i, p):
            off = pl.multiple_of(i * chunk, 8)
            pltpu.async_copy(spmem.at[sid, p],
                             out_hbm.at[pl.ds(base + off, chunk)], ssems[p])

        def s_wait(i, p):
            off = pl.multiple_of(i * chunk, 8)
            pltpu.make_async_copy(
                spmem.at[sid, p], out_hbm.at[pl.ds(base + off, chunk)],
                ssems[p]).wait()

        g_start(0, 0)
        g_start(1, 1)

        def emit_slot(i, b):
            p = b % 2
            g_wait(i, b)
            s_wait(i - 2, p)
            c_start(b, p)
            c_wait((b - 1) % nbuf, (p - 1) % 2)
            s_start(i - 1, (p - 1) % 2)
            g_start(i + 2, (b + 2) % nbuf)

        g_wait(0, 0)
        c_start(0, 0)
        g_start(2, 2)
        g_wait(1, 1)
        c_start(1, 1)
        c_wait(0, 0)
        s_start(0, 0)
        g_start(3, 3)

        n_steady = (n_chunk - 2 - 2) // nbuf
        def body(g, carry):
            for j in range(nbuf):
                emit_slot(2 + g * nbuf + j, (2 + j) % nbuf)
            return carry
        lax.fori_loop(0, n_steady, body, 0)
        for i in range(2 + nbuf * n_steady, n_chunk - 2):
            emit_slot(i, i % nbuf)

        for i in range(n_chunk - 2, n_chunk):
            b = i % nbuf
            p = b % 2
            g_wait(i, b)
            s_wait(i - 2, p)
            c_start(b, p)
            c_wait((b - 1) % nbuf, (p - 1) % 2)
            s_start(i - 1, (p - 1) % 2)
        b = (n_chunk - 1) % nbuf
        p = b % 2
        c_wait(b, p)
        s_start(n_chunk - 1, p)
        s_wait(n_chunk - 2, (p - 1) % 2)
        s_wait(n_chunk - 1, p)

    return gather_kernel


def kernel(token_ids, table):
    ids_flat = token_ids.reshape(-1).astype(jnp.int32)
    out = _build_gather(ids_flat.shape[0])(ids_flat, table)
    return out.reshape(token_ids.shape + (_EMBED,))
